# baseline (speedup 1.0000x reference)
import jax
import jax.numpy as jnp
from jax.experimental import pallas as pl
from jax.experimental.pallas import tpu as pltpu

_TN = 512


def _nnd_body(x_ref, yt_ref, d1_ref, d2_ref):
    i = pl.program_id(1)
    x = x_ref[0]
    y = yt_ref[0]
    xb = x.astype(jnp.bfloat16).astype(jnp.float32) * -2.0
    yb = y.astype(jnp.bfloat16).astype(jnp.float32)
    x2 = (x[:, 0:1] * x[:, 0:1] + x[:, 1:2] * x[:, 1:2]
          + x[:, 2:3] * x[:, 2:3])
    y2 = (y[0:1, :] * y[0:1, :] + y[1:2, :] * y[1:2, :]
          + y[2:3, :] * y[2:3, :])
    d = x2 + y2
    d += xb[:, 0:1] * yb[0:1, :]
    d += xb[:, 1:2] * yb[1:2, :]
    d += xb[:, 2:3] * yb[2:3, :]
    d1_ref[0] = jnp.min(d, axis=1, keepdims=True)
    m2 = jnp.min(d, axis=0, keepdims=True)

    @pl.when(i == 0)
    def _init():
        d2_ref[0] = m2

    @pl.when(i > 0)
    def _acc():
        d2_ref[0] = jnp.minimum(d2_ref[0], m2)


def _nnd_tc(x, y):
    B, N, _ = x.shape
    M = y.shape[1]
    yt = jnp.transpose(y, (0, 2, 1))
    d1, d2 = pl.pallas_call(
        _nnd_body,
        grid=(B, N // _TN),
        in_specs=[
            pl.BlockSpec((1, _TN, 3), lambda b, i: (b, i, 0)),
            pl.BlockSpec((1, 3, M), lambda b, i: (b, 0, 0)),
        ],
        out_specs=[
            pl.BlockSpec((1, _TN, 1), lambda b, i: (b, i, 0)),
            pl.BlockSpec((1, 1, M), lambda b, i: (b, 0, 0)),
        ],
        out_shape=[
            jax.ShapeDtypeStruct((B, N, 1), jnp.float32),
            jax.ShapeDtypeStruct((B, 1, M), jnp.float32),
        ],
    )(x, yt)
    return d1[:, :, 0], d2[:, 0, :]


def kernel(input1, input2):
    return _nnd_tc(input1, input2)


# speedup vs baseline: 2.2911x; 2.2911x over previous
import jax
import jax.numpy as jnp
from jax.experimental import pallas as pl
from jax.experimental.pallas import tpu as pltpu

_TN = 2048


def _nnd_body(x_ref, yt_ref, d1_ref, d2_ref):
    i = pl.program_id(1)
    x = x_ref[0]
    y = yt_ref[0]
    f32, bf16 = jnp.float32, jnp.bfloat16
    tn = x.shape[0]
    m = y.shape[1]
    x2 = (x[:, 0:1] * x[:, 0:1] + x[:, 1:2] * x[:, 1:2]
          + x[:, 2:3] * x[:, 2:3])
    y2 = (y[0:1, :] * y[0:1, :] + y[1:2, :] * y[1:2, :]
          + y[2:3, :] * y[2:3, :])
    x2h = x2.astype(bf16)
    x2l = (x2 - x2h.astype(f32)).astype(bf16)
    y2h = y2.astype(bf16)
    y2l = (y2 - y2h.astype(f32)).astype(bf16)
    one_c = jnp.ones((tn, 1), bf16)
    one_r = jnp.ones((1, m), bf16)
    xa = jnp.concatenate(
        [(x * -2.0).astype(bf16), x2h, x2l, one_c, one_c], axis=1)
    ya = jnp.concatenate(
        [y.astype(bf16), one_r, one_r, y2h, y2l], axis=0)
    d = jax.lax.dot_general(xa, ya, (((1,), (0,)), ((), ())),
                            preferred_element_type=jnp.float32)
    d1_ref[0] = jnp.min(d, axis=1, keepdims=True)
    m2 = jnp.min(d, axis=0, keepdims=True)

    @pl.when(i == 0)
    def _init():
        d2_ref[0] = m2

    @pl.when(i > 0)
    def _acc():
        d2_ref[0] = jnp.minimum(d2_ref[0], m2)


def _nnd_tc(x, y):
    B, N, _ = x.shape
    M = y.shape[1]
    yt = jnp.transpose(y, (0, 2, 1))
    d1, d2 = pl.pallas_call(
        _nnd_body,
        grid=(B, N // _TN),
        in_specs=[
            pl.BlockSpec((1, _TN, 3), lambda b, i: (b, i, 0)),
            pl.BlockSpec((1, 3, M), lambda b, i: (b, 0, 0)),
        ],
        out_specs=[
            pl.BlockSpec((1, _TN, 1), lambda b, i: (b, i, 0)),
            pl.BlockSpec((1, 1, M), lambda b, i: (b, 0, 0)),
        ],
        out_shape=[
            jax.ShapeDtypeStruct((B, N, 1), jnp.float32),
            jax.ShapeDtypeStruct((B, 1, M), jnp.float32),
        ],
    )(x, yt)
    return d1[:, :, 0], d2[:, 0, :]


def kernel(input1, input2):
    return _nnd_tc(input1, input2)
